# Initial kernel scaffold; baseline (speedup 1.0000x reference)
import jax
import jax.numpy as jnp
from jax.experimental import pallas as pl


def kernel(arm_loc_pred, arm_conf_pred, odm_loc_pred, odm_conf_pred, priors, targets):
    raise NotImplementedError("write your pallas kernel here")


# speedup vs baseline: 17.6284x; 17.6284x over previous
import functools

import jax
import jax.numpy as jnp
from jax.experimental import pallas as pl

_NUM_CLASSES = 21
_OVERLAP_THRESH = 0.5
_NEG_POS_RATIO = 3
_POS_PRIOR_THRESHOLD = 0.01
_NUM_OBJ = 8
_BISECT_ITERS = 40


def _row_kernel(arm_loc_ref, arm_conf_ref, odm_loc_ref, odm_conf_ref,
                priors_ref, targets_ref, out_ref):
    P = arm_loc_ref.shape[-1]

    al = arm_loc_ref[0]
    pcx = priors_ref[0:1, :]
    pcy = priors_ref[1:2, :]
    pw = priors_ref[2:3, :]
    ph = priors_ref[3:4, :]
    cx = pcx + al[0:1, :] * (0.1 * pw)
    cy = pcy + al[1:2, :] * (0.1 * ph)
    w = pw * jnp.exp(al[2:3, :] * 0.2)
    h = ph * jnp.exp(al[3:4, :] * 0.2)
    rx1 = cx - w * 0.5
    ry1 = cy - h * 0.5
    rx2 = cx + w * 0.5
    ry2 = cy + h * 0.5

    tgt = targets_ref[0]
    tx1 = tgt[:, 0:1]
    ty1 = tgt[:, 1:2]
    tx2 = tgt[:, 2:3]
    ty2 = tgt[:, 3:4]
    ix = jnp.maximum(jnp.minimum(tx2, rx2) - jnp.maximum(tx1, rx1), 0.0)
    iy = jnp.maximum(jnp.minimum(ty2, ry2) - jnp.maximum(ty1, ry1), 0.0)
    inter = ix * iy
    area_t = (tx2 - tx1) * (ty2 - ty1)
    area_p = w * h
    ov = inter / (area_t + area_p - inter)

    bto = jnp.max(ov, axis=0, keepdims=True)
    iota8 = jax.lax.broadcasted_iota(jnp.int32, ov.shape, 0)
    bti = jnp.min(jnp.where(ov == bto, iota8, _NUM_OBJ), axis=0,
                  keepdims=True)

    iota_p = jax.lax.broadcasted_iota(jnp.int32, (1, P), 1)
    for i in range(_NUM_OBJ):
        ov_i = ov[i:i + 1, :]
        m_i = jnp.max(ov_i)
        j_i = jnp.min(jnp.where(ov_i == m_i, iota_p, P))
        hit = iota_p == j_i
        bto = jnp.where(hit, 2.0, bto)
        bti = jnp.where(hit, i, bti)

    mx1 = jnp.zeros((1, P), jnp.float32)
    my1 = jnp.zeros((1, P), jnp.float32)
    mx2 = jnp.zeros((1, P), jnp.float32)
    my2 = jnp.zeros((1, P), jnp.float32)
    lab = jnp.zeros((1, P), jnp.float32)
    for i in range(_NUM_OBJ):
        sel = bti == i
        mx1 = jnp.where(sel, tgt[i, 0], mx1)
        my1 = jnp.where(sel, tgt[i, 1], my1)
        mx2 = jnp.where(sel, tgt[i, 2], mx2)
        my2 = jnp.where(sel, tgt[i, 3], my2)
        lab = jnp.where(sel, tgt[i, 4], lab)

    conf = jnp.where(bto < _OVERLAP_THRESH, 0.0, lab)
    conf_i = conf.astype(jnp.int32)
    pos = conf_i > 0

    g_cx = ((mx1 + mx2) * 0.5 - cx) / (0.1 * w)
    g_cy = ((my1 + my2) * 0.5 - cy) / (0.1 * h)
    g_w = jnp.log((mx2 - mx1) / w) / 0.2
    g_h = jnp.log((my2 - my1) / h) / 0.2
    ol = odm_loc_ref[0]

    def _sl1(d):
        ad = jnp.abs(d)
        return jnp.where(ad < 1.0, 0.5 * d * d, ad - 0.5)

    sl1 = (_sl1(ol[0:1, :] - g_cx) + _sl1(ol[1:2, :] - g_cy)
           + _sl1(ol[2:3, :] - g_w) + _sl1(ol[3:4, :] - g_h))
    loss_l = jnp.sum(jnp.where(pos, sl1, 0.0))

    oc = odm_conf_ref[0]
    mx = jnp.max(oc, axis=0, keepdims=True)
    lse = mx + jnp.log(jnp.sum(jnp.exp(oc - mx), axis=0, keepdims=True))
    iota_c = jax.lax.broadcasted_iota(jnp.int32, oc.shape, 0)
    gathered = jnp.sum(jnp.where(iota_c == conf_i, oc, 0.0), axis=0,
                       keepdims=True)
    ce = lse - gathered

    ac = arm_conf_ref[0]
    s1 = jax.nn.sigmoid(ac[1:2, :] - ac[0:1, :])
    ignore = jnp.logical_and(conf_i <= 0, s1 < _POS_PRIOR_THRESHOLD)
    proxy = jnp.where(jnp.logical_or(pos, ignore), 0.0, ce)

    num_pos = jnp.sum(pos.astype(jnp.float32))
    max_neg = jnp.sum((proxy > 0.0).astype(jnp.float32))
    k = jnp.minimum(_NEG_POS_RATIO * num_pos, max_neg)

    maxv = jnp.max(proxy)

    def _bisect(_, carry):
        lo, hi = carry
        mid = 0.5 * (lo + hi)
        cnt = jnp.sum((proxy >= mid).astype(jnp.float32))
        ok = cnt >= k
        return jnp.where(ok, mid, lo), jnp.where(ok, hi, mid)

    lo, _ = jax.lax.fori_loop(0, _BISECT_ITERS, _bisect,
                              (jnp.float32(0.0), maxv))
    ge = proxy >= lo
    cnt_lo = jnp.sum(ge.astype(jnp.float32))
    sum_lo = jnp.sum(jnp.where(ge, proxy, 0.0))
    topk = sum_lo - (cnt_lo - k) * lo
    topk = jnp.where(k > 0.0, topk, 0.0)

    ce_pos = jnp.sum(jnp.where(pos, ce, 0.0))
    loss_c = ce_pos + topk

    r = jax.lax.broadcasted_iota(jnp.int32, (8, 128), 0)
    c = jax.lax.broadcasted_iota(jnp.int32, (8, 128), 1)
    first = r == 0
    tile = jnp.where(jnp.logical_and(first, c == 0), loss_l, 0.0)
    tile = jnp.where(jnp.logical_and(first, c == 1), loss_c, tile)
    tile = jnp.where(jnp.logical_and(first, c == 2), num_pos, tile)
    out_ref[0] = tile


@jax.jit
def kernel(arm_loc_pred, arm_conf_pred, odm_loc_pred, odm_conf_pred, priors,
           targets):
    B, P, _ = arm_loc_pred.shape
    arm_loc_t = jnp.transpose(arm_loc_pred, (0, 2, 1))
    arm_conf_t = jnp.transpose(arm_conf_pred, (0, 2, 1))
    odm_loc_t = jnp.transpose(odm_loc_pred, (0, 2, 1))
    odm_conf_t = jnp.transpose(odm_conf_pred, (0, 2, 1))
    priors_t = jnp.transpose(priors, (1, 0))

    out = pl.pallas_call(
        _row_kernel,
        grid=(B,),
        in_specs=[
            pl.BlockSpec((1, 4, P), lambda b: (b, 0, 0)),
            pl.BlockSpec((1, 2, P), lambda b: (b, 0, 0)),
            pl.BlockSpec((1, 4, P), lambda b: (b, 0, 0)),
            pl.BlockSpec((1, _NUM_CLASSES, P), lambda b: (b, 0, 0)),
            pl.BlockSpec((4, P), lambda b: (0, 0)),
            pl.BlockSpec((1, _NUM_OBJ, 5), lambda b: (b, 0, 0)),
        ],
        out_specs=pl.BlockSpec((1, 8, 128), lambda b: (b, 0, 0)),
        out_shape=jax.ShapeDtypeStruct((B, 8, 128), jnp.float32),
    )(arm_loc_t, arm_conf_t, odm_loc_t, odm_conf_t, priors_t, targets)

    loss_l = jnp.sum(out[:, 0, 0])
    loss_c = jnp.sum(out[:, 0, 1])
    total = jnp.sum(out[:, 0, 2])
    return (loss_l / total, loss_c / total)


# speedup vs baseline: 27.8151x; 1.5779x over previous
import jax
import jax.numpy as jnp
from jax.experimental import pallas as pl

_NUM_CLASSES = 21
_OVERLAP_THRESH = 0.5
_NEG_POS_RATIO = 3
_POS_PRIOR_THRESHOLD = 0.01
_NUM_OBJ = 8
_BISECT_ITERS = 40
_S = 128
_P_PAD = _S * _S


def _row_kernel(arm_loc_ref, arm_conf_ref, odm_loc_ref, odm_conf_ref,
                priors_ref, targets_ref, valid_p, out_ref):
    T = (_S, _S)

    al = arm_loc_ref[0]
    pcx = priors_ref[0]
    pcy = priors_ref[1]
    pw = priors_ref[2]
    ph = priors_ref[3]
    cx = pcx + al[0] * (0.1 * pw)
    cy = pcy + al[1] * (0.1 * ph)
    w = pw * jnp.exp(al[2] * 0.2)
    h = ph * jnp.exp(al[3] * 0.2)
    rx1 = cx - w * 0.5
    ry1 = cy - h * 0.5
    rx2 = cx + w * 0.5
    ry2 = cy + h * 0.5

    idx2d = (jax.lax.broadcasted_iota(jnp.int32, T, 0) * _S
             + jax.lax.broadcasted_iota(jnp.int32, T, 1))

    tgt = targets_ref[0]

    bto = jnp.zeros(T, jnp.float32)
    bti = jnp.zeros(T, jnp.int32)
    best_prior = []
    for i in range(_NUM_OBJ):
        tx1 = tgt[i:i + 1, 0:1]
        ty1 = tgt[i:i + 1, 1:2]
        tx2 = tgt[i:i + 1, 2:3]
        ty2 = tgt[i:i + 1, 3:4]
        ix = jnp.maximum(jnp.minimum(tx2, rx2) - jnp.maximum(tx1, rx1), 0.0)
        iy = jnp.maximum(jnp.minimum(ty2, ry2) - jnp.maximum(ty1, ry1), 0.0)
        inter = ix * iy
        area_t = (tx2 - tx1) * (ty2 - ty1)
        ov = inter / (area_t + w * h - inter)
        if i == 0:
            bto = ov
        else:
            upd = ov > bto
            bto = jnp.maximum(bto, ov)
            bti = jnp.where(upd, i, bti)
        m_i = jnp.max(ov)
        best_prior.append(jnp.min(jnp.where(ov == m_i, idx2d, _P_PAD)))

    for i in range(_NUM_OBJ):
        hit = idx2d == best_prior[i]
        bto = jnp.where(hit, 2.0, bto)
        bti = jnp.where(hit, i, bti)

    mx1 = jnp.zeros(T, jnp.float32)
    my1 = jnp.zeros(T, jnp.float32)
    mx2 = jnp.zeros(T, jnp.float32)
    my2 = jnp.zeros(T, jnp.float32)
    lab = jnp.zeros(T, jnp.float32)
    for i in range(_NUM_OBJ):
        sel = bti == i
        mx1 = jnp.where(sel, tgt[i, 0], mx1)
        my1 = jnp.where(sel, tgt[i, 1], my1)
        mx2 = jnp.where(sel, tgt[i, 2], mx2)
        my2 = jnp.where(sel, tgt[i, 3], my2)
        lab = jnp.where(sel, tgt[i, 4], lab)

    conf = jnp.where(bto < _OVERLAP_THRESH, 0.0, lab)
    conf_i = conf.astype(jnp.int32)
    pos = conf_i > 0

    g_cx = ((mx1 + mx2) * 0.5 - cx) / (0.1 * w)
    g_cy = ((my1 + my2) * 0.5 - cy) / (0.1 * h)
    g_w = jnp.log((mx2 - mx1) / w) / 0.2
    g_h = jnp.log((my2 - my1) / h) / 0.2
    ol = odm_loc_ref[0]

    def _sl1(d):
        ad = jnp.abs(d)
        return jnp.where(ad < 1.0, 0.5 * d * d, ad - 0.5)

    sl1 = (_sl1(ol[0] - g_cx) + _sl1(ol[1] - g_cy)
           + _sl1(ol[2] - g_w) + _sl1(ol[3] - g_h))
    loss_l = jnp.sum(jnp.where(pos, sl1, 0.0))

    oc = odm_conf_ref[0]
    mx = jnp.max(oc, axis=0)
    lse = mx + jnp.log(jnp.sum(jnp.exp(oc - mx[None]), axis=0))
    iota_c = jax.lax.broadcasted_iota(jnp.int32, oc.shape, 0)
    gathered = jnp.sum(jnp.where(iota_c == conf_i[None], oc, 0.0), axis=0)
    ce = lse - gathered

    ac = arm_conf_ref[0]
    s1 = jax.nn.sigmoid(ac[1] - ac[0])
    keep = jnp.logical_or(pos, jnp.logical_and(conf_i <= 0,
                                               s1 < _POS_PRIOR_THRESHOLD))
    proxy = jnp.where(jnp.logical_or(keep, jnp.logical_not(valid_p)),
                      0.0, ce)

    num_pos = jnp.sum(pos.astype(jnp.float32))
    max_neg = jnp.sum((proxy > 0.0).astype(jnp.float32))
    k = jnp.minimum(_NEG_POS_RATIO * num_pos, max_neg)

    maxv = jnp.max(proxy)

    def _bisect(_, carry):
        lo, hi = carry
        mid = 0.5 * (lo + hi)
        cnt = jnp.sum((proxy >= mid).astype(jnp.float32))
        ok = cnt >= k
        return jnp.where(ok, mid, lo), jnp.where(ok, hi, mid)

    lo, _ = jax.lax.fori_loop(0, _BISECT_ITERS, _bisect,
                              (jnp.float32(0.0), maxv))
    ge = proxy >= lo
    cnt_lo = jnp.sum(ge.astype(jnp.float32))
    sum_lo = jnp.sum(jnp.where(ge, proxy, 0.0))
    topk = sum_lo - (cnt_lo - k) * lo
    topk = jnp.where(k > 0.0, topk, 0.0)

    ce_pos = jnp.sum(jnp.where(pos, ce, 0.0))
    loss_c = ce_pos + topk

    r = jax.lax.broadcasted_iota(jnp.int32, (8, 128), 0)
    c = jax.lax.broadcasted_iota(jnp.int32, (8, 128), 1)
    first = r == 0
    tile = jnp.where(jnp.logical_and(first, c == 0), loss_l, 0.0)
    tile = jnp.where(jnp.logical_and(first, c == 1), loss_c, tile)
    tile = jnp.where(jnp.logical_and(first, c == 2), num_pos, tile)
    out_ref[0] = tile


def _to_tiles(x):
    B, P, C = x.shape
    xt = jnp.transpose(x, (0, 2, 1))
    xt = jnp.pad(xt, ((0, 0), (0, 0), (0, _P_PAD - P)))
    return xt.reshape(B, C, _S, _S)


@jax.jit
def kernel(arm_loc_pred, arm_conf_pred, odm_loc_pred, odm_conf_pred, priors,
           targets):
    B, P, _ = arm_loc_pred.shape
    priors_t = jnp.pad(jnp.transpose(priors, (1, 0)),
                       ((0, 0), (0, _P_PAD - P))).reshape(4, _S, _S)

    def body(*refs):
        P_idx = (jax.lax.broadcasted_iota(jnp.int32, (_S, _S), 0) * _S
                 + jax.lax.broadcasted_iota(jnp.int32, (_S, _S), 1))
        _row_kernel(*refs[:6], P_idx < P, refs[6])

    out = pl.pallas_call(
        body,
        grid=(B,),
        in_specs=[
            pl.BlockSpec((1, 4, _S, _S), lambda b: (b, 0, 0, 0)),
            pl.BlockSpec((1, 2, _S, _S), lambda b: (b, 0, 0, 0)),
            pl.BlockSpec((1, 4, _S, _S), lambda b: (b, 0, 0, 0)),
            pl.BlockSpec((1, _NUM_CLASSES, _S, _S), lambda b: (b, 0, 0, 0)),
            pl.BlockSpec((4, _S, _S), lambda b: (0, 0, 0)),
            pl.BlockSpec((1, _NUM_OBJ, 5), lambda b: (b, 0, 0)),
        ],
        out_specs=pl.BlockSpec((1, 8, 128), lambda b: (b, 0, 0)),
        out_shape=jax.ShapeDtypeStruct((B, 8, 128), jnp.float32),
    )(_to_tiles(arm_loc_pred), _to_tiles(arm_conf_pred),
      _to_tiles(odm_loc_pred), _to_tiles(odm_conf_pred), priors_t, targets)

    loss_l = jnp.sum(out[:, 0, 0])
    loss_c = jnp.sum(out[:, 0, 1])
    total = jnp.sum(out[:, 0, 2])
    return (loss_l / total, loss_c / total)


# speedup vs baseline: 34.1792x; 1.2288x over previous
import jax
import jax.numpy as jnp
from jax.experimental import pallas as pl
from jax.experimental.pallas import tpu as pltpu

_NUM_CLASSES = 21
_OVERLAP_THRESH = 0.5
_NEG_POS_RATIO = 3
_POS_PRIOR_THRESHOLD = 0.01
_NUM_OBJ = 8
_BISECT_ITERS = 20
_S = 128
_P_PAD = _S * _S


def _row_kernel(arm_loc_ref, arm_conf_ref, odm_loc_ref, odm_conf_ref,
                priors_ref, targets_ref, valid_p, out_ref):
    T = (_S, _S)

    al = arm_loc_ref[0]
    pcx = priors_ref[0]
    pcy = priors_ref[1]
    pw = priors_ref[2]
    ph = priors_ref[3]
    cx = pcx + al[0] * (0.1 * pw)
    cy = pcy + al[1] * (0.1 * ph)
    w = pw * jnp.exp(al[2] * 0.2)
    h = ph * jnp.exp(al[3] * 0.2)
    rx1 = cx - w * 0.5
    ry1 = cy - h * 0.5
    rx2 = cx + w * 0.5
    ry2 = cy + h * 0.5

    idx2d = (jax.lax.broadcasted_iota(jnp.int32, T, 0) * _S
             + jax.lax.broadcasted_iota(jnp.int32, T, 1))

    tgt = targets_ref[0]

    bto = jnp.zeros(T, jnp.float32)
    bti = jnp.zeros(T, jnp.int32)
    best_prior = []
    for i in range(_NUM_OBJ):
        tx1 = tgt[i:i + 1, 0:1]
        ty1 = tgt[i:i + 1, 1:2]
        tx2 = tgt[i:i + 1, 2:3]
        ty2 = tgt[i:i + 1, 3:4]
        ix = jnp.maximum(jnp.minimum(tx2, rx2) - jnp.maximum(tx1, rx1), 0.0)
        iy = jnp.maximum(jnp.minimum(ty2, ry2) - jnp.maximum(ty1, ry1), 0.0)
        inter = ix * iy
        area_t = (tx2 - tx1) * (ty2 - ty1)
        ov = inter / (area_t + w * h - inter)
        if i == 0:
            bto = ov
        else:
            upd = ov > bto
            bto = jnp.maximum(bto, ov)
            bti = jnp.where(upd, i, bti)
        m_i = jnp.max(ov)
        best_prior.append(jnp.min(jnp.where(ov == m_i, idx2d, _P_PAD)))

    for i in range(_NUM_OBJ):
        hit = idx2d == best_prior[i]
        bto = jnp.where(hit, 2.0, bto)
        bti = jnp.where(hit, i, bti)

    mx1 = jnp.zeros(T, jnp.float32)
    my1 = jnp.zeros(T, jnp.float32)
    mx2 = jnp.zeros(T, jnp.float32)
    my2 = jnp.zeros(T, jnp.float32)
    lab = jnp.zeros(T, jnp.float32)
    for i in range(_NUM_OBJ):
        sel = bti == i
        mx1 = jnp.where(sel, tgt[i, 0], mx1)
        my1 = jnp.where(sel, tgt[i, 1], my1)
        mx2 = jnp.where(sel, tgt[i, 2], mx2)
        my2 = jnp.where(sel, tgt[i, 3], my2)
        lab = jnp.where(sel, tgt[i, 4], lab)

    conf = jnp.where(bto < _OVERLAP_THRESH, 0.0, lab)
    conf_i = conf.astype(jnp.int32)
    pos = conf_i > 0

    g_cx = ((mx1 + mx2) * 0.5 - cx) / (0.1 * w)
    g_cy = ((my1 + my2) * 0.5 - cy) / (0.1 * h)
    g_w = jnp.log((mx2 - mx1) / w) / 0.2
    g_h = jnp.log((my2 - my1) / h) / 0.2
    ol = odm_loc_ref[0]

    def _sl1(d):
        ad = jnp.abs(d)
        return jnp.where(ad < 1.0, 0.5 * d * d, ad - 0.5)

    sl1 = (_sl1(ol[0] - g_cx) + _sl1(ol[1] - g_cy)
           + _sl1(ol[2] - g_w) + _sl1(ol[3] - g_h))
    loss_l = jnp.sum(jnp.where(pos, sl1, 0.0))

    oc = odm_conf_ref[0]
    mx = jnp.max(oc, axis=0)
    lse = mx + jnp.log(jnp.sum(jnp.exp(oc - mx[None]), axis=0))
    iota_c = jax.lax.broadcasted_iota(jnp.int32, oc.shape, 0)
    gathered = jnp.sum(jnp.where(iota_c == conf_i[None], oc, 0.0), axis=0)
    ce = lse - gathered

    ac = arm_conf_ref[0]
    s1 = jax.nn.sigmoid(ac[1] - ac[0])
    keep = jnp.logical_or(pos, jnp.logical_and(conf_i <= 0,
                                               s1 < _POS_PRIOR_THRESHOLD))
    proxy = jnp.where(jnp.logical_or(keep, jnp.logical_not(valid_p)),
                      0.0, ce)

    num_pos = jnp.sum(pos.astype(jnp.float32))
    max_neg = jnp.sum((proxy > 0.0).astype(jnp.float32))
    k = jnp.minimum(_NEG_POS_RATIO * num_pos, max_neg)

    maxv = jnp.max(proxy)

    def _bisect(_, carry):
        lo, hi = carry
        mid = 0.5 * (lo + hi)
        cnt = jnp.sum((proxy >= mid).astype(jnp.float32))
        ok = cnt >= k
        return jnp.where(ok, mid, lo), jnp.where(ok, hi, mid)

    lo, _ = jax.lax.fori_loop(0, _BISECT_ITERS, _bisect,
                              (jnp.float32(0.0), maxv))
    ge = proxy >= lo
    cnt_lo = jnp.sum(ge.astype(jnp.float32))
    sum_lo = jnp.sum(jnp.where(ge, proxy, 0.0))
    topk = sum_lo - (cnt_lo - k) * lo
    topk = jnp.where(k > 0.0, topk, 0.0)

    ce_pos = jnp.sum(jnp.where(pos, ce, 0.0))
    loss_c = ce_pos + topk

    r = jax.lax.broadcasted_iota(jnp.int32, (8, 128), 0)
    c = jax.lax.broadcasted_iota(jnp.int32, (8, 128), 1)
    first = r == 0
    tile = jnp.where(jnp.logical_and(first, c == 0), loss_l, 0.0)
    tile = jnp.where(jnp.logical_and(first, c == 1), loss_c, tile)
    tile = jnp.where(jnp.logical_and(first, c == 2), num_pos, tile)
    out_ref[0] = tile


def _to_tiles(x):
    B, P, C = x.shape
    xt = jnp.transpose(x, (0, 2, 1))
    xt = jnp.pad(xt, ((0, 0), (0, 0), (0, _P_PAD - P)))
    return xt.reshape(B, C, _S, _S)


@jax.jit
def kernel(arm_loc_pred, arm_conf_pred, odm_loc_pred, odm_conf_pred, priors,
           targets):
    B, P, _ = arm_loc_pred.shape
    priors_t = jnp.pad(jnp.transpose(priors, (1, 0)),
                       ((0, 0), (0, _P_PAD - P))).reshape(4, _S, _S)

    def body(*refs):
        P_idx = (jax.lax.broadcasted_iota(jnp.int32, (_S, _S), 0) * _S
                 + jax.lax.broadcasted_iota(jnp.int32, (_S, _S), 1))
        _row_kernel(*refs[:6], P_idx < P, refs[6])

    out = pl.pallas_call(
        body,
        grid=(B,),
        in_specs=[
            pl.BlockSpec((1, 4, _S, _S), lambda b: (b, 0, 0, 0)),
            pl.BlockSpec((1, 2, _S, _S), lambda b: (b, 0, 0, 0)),
            pl.BlockSpec((1, 4, _S, _S), lambda b: (b, 0, 0, 0)),
            pl.BlockSpec((1, _NUM_CLASSES, _S, _S), lambda b: (b, 0, 0, 0)),
            pl.BlockSpec((4, _S, _S), lambda b: (0, 0, 0)),
            pl.BlockSpec((1, _NUM_OBJ, 5), lambda b: (b, 0, 0)),
        ],
        out_specs=pl.BlockSpec((1, 8, 128), lambda b: (b, 0, 0)),
        out_shape=jax.ShapeDtypeStruct((B, 8, 128), jnp.float32),
        compiler_params=pltpu.CompilerParams(
            dimension_semantics=("parallel",)),
    )(_to_tiles(arm_loc_pred), _to_tiles(arm_conf_pred),
      _to_tiles(odm_loc_pred), _to_tiles(odm_conf_pred), priors_t, targets)

    loss_l = jnp.sum(out[:, 0, 0])
    loss_c = jnp.sum(out[:, 0, 1])
    total = jnp.sum(out[:, 0, 2])
    return (loss_l / total, loss_c / total)


# speedup vs baseline: 34.4685x; 1.0085x over previous
import jax
import jax.numpy as jnp
from jax.experimental import pallas as pl
from jax.experimental.pallas import tpu as pltpu

_NUM_CLASSES = 21
_OVERLAP_THRESH = 0.5
_NEG_POS_RATIO = 3
_POS_PRIOR_THRESHOLD = 0.01
_NUM_OBJ = 8
_BISECT_ITERS = 20
_S = 128
_P_PAD = _S * _S
_ROWS_PER_PROG = 2


def _row_kernel(r, arm_loc_ref, arm_conf_ref, odm_loc_ref, odm_conf_ref,
                priors_ref, targets_ref, valid_p, out_ref):
    T = (_S, _S)

    al = arm_loc_ref[r]
    pcx = priors_ref[0]
    pcy = priors_ref[1]
    pw = priors_ref[2]
    ph = priors_ref[3]
    cx = pcx + al[0] * (0.1 * pw)
    cy = pcy + al[1] * (0.1 * ph)
    w = pw * jnp.exp(al[2] * 0.2)
    h = ph * jnp.exp(al[3] * 0.2)
    rx1 = cx - w * 0.5
    ry1 = cy - h * 0.5
    rx2 = cx + w * 0.5
    ry2 = cy + h * 0.5

    idx2d = (jax.lax.broadcasted_iota(jnp.int32, T, 0) * _S
             + jax.lax.broadcasted_iota(jnp.int32, T, 1))

    tgt = targets_ref[r]

    bto = jnp.zeros(T, jnp.float32)
    bti = jnp.zeros(T, jnp.int32)
    best_prior = []
    for i in range(_NUM_OBJ):
        tx1 = tgt[i:i + 1, 0:1]
        ty1 = tgt[i:i + 1, 1:2]
        tx2 = tgt[i:i + 1, 2:3]
        ty2 = tgt[i:i + 1, 3:4]
        ix = jnp.maximum(jnp.minimum(tx2, rx2) - jnp.maximum(tx1, rx1), 0.0)
        iy = jnp.maximum(jnp.minimum(ty2, ry2) - jnp.maximum(ty1, ry1), 0.0)
        inter = ix * iy
        area_t = (tx2 - tx1) * (ty2 - ty1)
        ov = inter / (area_t + w * h - inter)
        if i == 0:
            bto = ov
        else:
            upd = ov > bto
            bto = jnp.maximum(bto, ov)
            bti = jnp.where(upd, i, bti)
        m_i = jnp.max(ov)
        best_prior.append(jnp.min(jnp.where(ov == m_i, idx2d, _P_PAD)))

    for i in range(_NUM_OBJ):
        hit = idx2d == best_prior[i]
        bto = jnp.where(hit, 2.0, bto)
        bti = jnp.where(hit, i, bti)

    mx1 = jnp.zeros(T, jnp.float32)
    my1 = jnp.zeros(T, jnp.float32)
    mx2 = jnp.zeros(T, jnp.float32)
    my2 = jnp.zeros(T, jnp.float32)
    lab = jnp.zeros(T, jnp.float32)
    for i in range(_NUM_OBJ):
        sel = bti == i
        mx1 = jnp.where(sel, tgt[i, 0], mx1)
        my1 = jnp.where(sel, tgt[i, 1], my1)
        mx2 = jnp.where(sel, tgt[i, 2], mx2)
        my2 = jnp.where(sel, tgt[i, 3], my2)
        lab = jnp.where(sel, tgt[i, 4], lab)

    conf = jnp.where(bto < _OVERLAP_THRESH, 0.0, lab)
    conf_i = conf.astype(jnp.int32)
    pos = conf_i > 0

    g_cx = ((mx1 + mx2) * 0.5 - cx) / (0.1 * w)
    g_cy = ((my1 + my2) * 0.5 - cy) / (0.1 * h)
    g_w = jnp.log((mx2 - mx1) / w) / 0.2
    g_h = jnp.log((my2 - my1) / h) / 0.2
    ol = odm_loc_ref[r]

    def _sl1(d):
        ad = jnp.abs(d)
        return jnp.where(ad < 1.0, 0.5 * d * d, ad - 0.5)

    sl1 = (_sl1(ol[0] - g_cx) + _sl1(ol[1] - g_cy)
           + _sl1(ol[2] - g_w) + _sl1(ol[3] - g_h))
    loss_l = jnp.sum(jnp.where(pos, sl1, 0.0))

    oc = odm_conf_ref[r]
    mx = jnp.max(oc, axis=0)
    lse = mx + jnp.log(jnp.sum(jnp.exp(oc - mx[None]), axis=0))
    iota_c = jax.lax.broadcasted_iota(jnp.int32, oc.shape, 0)
    gathered = jnp.sum(jnp.where(iota_c == conf_i[None], oc, 0.0), axis=0)
    ce = lse - gathered

    ac = arm_conf_ref[r]
    s1 = jax.nn.sigmoid(ac[1] - ac[0])
    keep = jnp.logical_or(pos, jnp.logical_and(conf_i <= 0,
                                               s1 < _POS_PRIOR_THRESHOLD))
    proxy = jnp.where(jnp.logical_or(keep, jnp.logical_not(valid_p)),
                      0.0, ce)

    num_pos = jnp.sum(pos.astype(jnp.float32))
    max_neg = jnp.sum((proxy > 0.0).astype(jnp.float32))
    k = jnp.minimum(_NEG_POS_RATIO * num_pos, max_neg)

    maxv = jnp.max(proxy)

    def _bisect(_, carry):
        lo, hi = carry
        mid = 0.5 * (lo + hi)
        cnt = jnp.sum((proxy >= mid).astype(jnp.float32))
        ok = cnt >= k
        return jnp.where(ok, mid, lo), jnp.where(ok, hi, mid)

    lo, _ = jax.lax.fori_loop(0, _BISECT_ITERS, _bisect,
                              (jnp.float32(0.0), maxv))
    ge = proxy >= lo
    cnt_lo = jnp.sum(ge.astype(jnp.float32))
    sum_lo = jnp.sum(jnp.where(ge, proxy, 0.0))
    topk = sum_lo - (cnt_lo - k) * lo
    topk = jnp.where(k > 0.0, topk, 0.0)

    ce_pos = jnp.sum(jnp.where(pos, ce, 0.0))
    loss_c = ce_pos + topk

    rr = jax.lax.broadcasted_iota(jnp.int32, (8, 128), 0)
    cc = jax.lax.broadcasted_iota(jnp.int32, (8, 128), 1)
    first = rr == 0
    tile = jnp.where(jnp.logical_and(first, cc == 0), loss_l, 0.0)
    tile = jnp.where(jnp.logical_and(first, cc == 1), loss_c, tile)
    tile = jnp.where(jnp.logical_and(first, cc == 2), num_pos, tile)
    out_ref[r] = tile


def _to_tiles(x):
    B, P, C = x.shape
    xt = jnp.transpose(x, (0, 2, 1))
    xt = jnp.pad(xt, ((0, 0), (0, 0), (0, _P_PAD - P)))
    return xt.reshape(B, C, _S, _S)


@jax.jit
def kernel(arm_loc_pred, arm_conf_pred, odm_loc_pred, odm_conf_pred, priors,
           targets):
    B, P, _ = arm_loc_pred.shape
    priors_t = jnp.pad(jnp.transpose(priors, (1, 0)),
                       ((0, 0), (0, _P_PAD - P))).reshape(4, _S, _S)

    def body(*refs):
        P_idx = (jax.lax.broadcasted_iota(jnp.int32, (_S, _S), 0) * _S
                 + jax.lax.broadcasted_iota(jnp.int32, (_S, _S), 1))
        valid = P_idx < P
        for r in range(_ROWS_PER_PROG):
            _row_kernel(r, *refs[:6], valid, refs[6])

    out = pl.pallas_call(
        body,
        grid=(B // _ROWS_PER_PROG,),
        in_specs=[
            pl.BlockSpec((_ROWS_PER_PROG, 4, _S, _S),
                         lambda b: (b, 0, 0, 0)),
            pl.BlockSpec((_ROWS_PER_PROG, 2, _S, _S),
                         lambda b: (b, 0, 0, 0)),
            pl.BlockSpec((_ROWS_PER_PROG, 4, _S, _S),
                         lambda b: (b, 0, 0, 0)),
            pl.BlockSpec((_ROWS_PER_PROG, _NUM_CLASSES, _S, _S),
                         lambda b: (b, 0, 0, 0)),
            pl.BlockSpec((4, _S, _S), lambda b: (0, 0, 0)),
            pl.BlockSpec((_ROWS_PER_PROG, _NUM_OBJ, 5), lambda b: (b, 0, 0)),
        ],
        out_specs=pl.BlockSpec((_ROWS_PER_PROG, 8, 128), lambda b: (b, 0, 0)),
        out_shape=jax.ShapeDtypeStruct((B, 8, 128), jnp.float32),
        compiler_params=pltpu.CompilerParams(
            dimension_semantics=("parallel",)),
    )(_to_tiles(arm_loc_pred), _to_tiles(arm_conf_pred),
      _to_tiles(odm_loc_pred), _to_tiles(odm_conf_pred), priors_t, targets)

    loss_l = jnp.sum(out[:, 0, 0])
    loss_c = jnp.sum(out[:, 0, 1])
    total = jnp.sum(out[:, 0, 2])
    return (loss_l / total, loss_c / total)


# speedup vs baseline: 34.4891x; 1.0006x over previous
import jax
import jax.numpy as jnp
from jax.experimental import pallas as pl
from jax.experimental.pallas import tpu as pltpu

_NUM_CLASSES = 21
_OVERLAP_THRESH = 0.5
_NEG_POS_RATIO = 3
_POS_PRIOR_THRESHOLD = 0.01
_NUM_OBJ = 8
_BISECT_ITERS = 20
_S = 128
_P_PAD = _S * _S
_ROWS_PER_PROG = 2


def _row_kernel(r, arm_loc_ref, arm_conf_ref, odm_loc_ref, odm_conf_ref,
                priors_ref, targets_ref, valid_p, out_ref):
    T = (_S, _S)

    al = arm_loc_ref[r]
    pcx = priors_ref[0]
    pcy = priors_ref[1]
    pw = priors_ref[2]
    ph = priors_ref[3]
    cx = pcx + al[0] * (0.1 * pw)
    cy = pcy + al[1] * (0.1 * ph)
    w = pw * jnp.exp(al[2] * 0.2)
    h = ph * jnp.exp(al[3] * 0.2)
    rx1 = cx - w * 0.5
    ry1 = cy - h * 0.5
    rx2 = cx + w * 0.5
    ry2 = cy + h * 0.5

    idx2d = (jax.lax.broadcasted_iota(jnp.int32, T, 0) * _S
             + jax.lax.broadcasted_iota(jnp.int32, T, 1))

    tgt = targets_ref[r]

    bto = jnp.zeros(T, jnp.float32)
    bti = jnp.zeros(T, jnp.int32)
    best_prior = []
    for i in range(_NUM_OBJ):
        tx1 = tgt[i:i + 1, 0:1]
        ty1 = tgt[i:i + 1, 1:2]
        tx2 = tgt[i:i + 1, 2:3]
        ty2 = tgt[i:i + 1, 3:4]
        ix = jnp.maximum(jnp.minimum(tx2, rx2) - jnp.maximum(tx1, rx1), 0.0)
        iy = jnp.maximum(jnp.minimum(ty2, ry2) - jnp.maximum(ty1, ry1), 0.0)
        inter = ix * iy
        area_t = (tx2 - tx1) * (ty2 - ty1)
        ov = inter / (area_t + w * h - inter)
        if i == 0:
            bto = ov
        else:
            upd = ov > bto
            bto = jnp.maximum(bto, ov)
            bti = jnp.where(upd, i, bti)
        m_i = jnp.max(ov)
        best_prior.append(jnp.min(jnp.where(ov == m_i, idx2d, _P_PAD)))

    for i in range(_NUM_OBJ):
        hit = idx2d == best_prior[i]
        bto = jnp.where(hit, 2.0, bto)
        bti = jnp.where(hit, i, bti)

    mx1 = jnp.zeros(T, jnp.float32)
    my1 = jnp.zeros(T, jnp.float32)
    mx2 = jnp.zeros(T, jnp.float32)
    my2 = jnp.zeros(T, jnp.float32)
    lab = jnp.zeros(T, jnp.float32)
    for i in range(_NUM_OBJ):
        sel = bti == i
        mx1 = jnp.where(sel, tgt[i, 0], mx1)
        my1 = jnp.where(sel, tgt[i, 1], my1)
        mx2 = jnp.where(sel, tgt[i, 2], mx2)
        my2 = jnp.where(sel, tgt[i, 3], my2)
        lab = jnp.where(sel, tgt[i, 4], lab)

    conf = jnp.where(bto < _OVERLAP_THRESH, 0.0, lab)
    conf_i = conf.astype(jnp.int32)
    pos = conf_i > 0

    g_cx = ((mx1 + mx2) * 0.5 - cx) / (0.1 * w)
    g_cy = ((my1 + my2) * 0.5 - cy) / (0.1 * h)
    g_w = jnp.log((mx2 - mx1) / w) / 0.2
    g_h = jnp.log((my2 - my1) / h) / 0.2
    ol = odm_loc_ref[r]

    def _sl1(d):
        ad = jnp.abs(d)
        return jnp.where(ad < 1.0, 0.5 * d * d, ad - 0.5)

    sl1 = (_sl1(ol[0] - g_cx) + _sl1(ol[1] - g_cy)
           + _sl1(ol[2] - g_w) + _sl1(ol[3] - g_h))
    loss_l = jnp.sum(jnp.where(pos, sl1, 0.0))

    oc = odm_conf_ref[r]
    mx = jnp.max(oc, axis=0)
    lse = mx + jnp.log(jnp.sum(jnp.exp(oc - mx[None]), axis=0))
    iota_c = jax.lax.broadcasted_iota(jnp.int32, oc.shape, 0)
    gathered = jnp.sum(jnp.where(iota_c == conf_i[None], oc, 0.0), axis=0)
    ce = lse - gathered

    ac = arm_conf_ref[r]
    s1 = jax.nn.sigmoid(ac[1] - ac[0])
    keep = jnp.logical_or(pos, jnp.logical_and(conf_i <= 0,
                                               s1 < _POS_PRIOR_THRESHOLD))
    proxy = jnp.where(jnp.logical_or(keep, jnp.logical_not(valid_p)),
                      0.0, ce)

    num_pos = jnp.sum(pos.astype(jnp.float32))
    max_neg = jnp.sum((proxy > 0.0).astype(jnp.float32))
    k = jnp.minimum(_NEG_POS_RATIO * num_pos, max_neg)

    maxv = jnp.max(proxy)

    lo = jnp.float32(0.0)
    hi = maxv
    for _ in range(_BISECT_ITERS):
        mid = 0.5 * (lo + hi)
        cnt = jnp.sum((proxy >= mid).astype(jnp.float32))
        ok = cnt >= k
        lo = jnp.where(ok, mid, lo)
        hi = jnp.where(ok, hi, mid)
    ge = proxy >= lo
    cnt_lo = jnp.sum(ge.astype(jnp.float32))
    sum_lo = jnp.sum(jnp.where(ge, proxy, 0.0))
    topk = sum_lo - (cnt_lo - k) * lo
    topk = jnp.where(k > 0.0, topk, 0.0)

    ce_pos = jnp.sum(jnp.where(pos, ce, 0.0))
    loss_c = ce_pos + topk

    rr = jax.lax.broadcasted_iota(jnp.int32, (8, 128), 0)
    cc = jax.lax.broadcasted_iota(jnp.int32, (8, 128), 1)
    first = rr == 0
    tile = jnp.where(jnp.logical_and(first, cc == 0), loss_l, 0.0)
    tile = jnp.where(jnp.logical_and(first, cc == 1), loss_c, tile)
    tile = jnp.where(jnp.logical_and(first, cc == 2), num_pos, tile)
    out_ref[r] = tile


def _to_tiles(x):
    B, P, C = x.shape
    xt = jnp.transpose(x, (0, 2, 1))
    xt = jnp.pad(xt, ((0, 0), (0, 0), (0, _P_PAD - P)))
    return xt.reshape(B, C, _S, _S)


@jax.jit
def kernel(arm_loc_pred, arm_conf_pred, odm_loc_pred, odm_conf_pred, priors,
           targets):
    B, P, _ = arm_loc_pred.shape
    priors_t = jnp.pad(jnp.transpose(priors, (1, 0)),
                       ((0, 0), (0, _P_PAD - P))).reshape(4, _S, _S)

    def body(*refs):
        P_idx = (jax.lax.broadcasted_iota(jnp.int32, (_S, _S), 0) * _S
                 + jax.lax.broadcasted_iota(jnp.int32, (_S, _S), 1))
        valid = P_idx < P
        for r in range(_ROWS_PER_PROG):
            _row_kernel(r, *refs[:6], valid, refs[6])

    out = pl.pallas_call(
        body,
        grid=(B // _ROWS_PER_PROG,),
        in_specs=[
            pl.BlockSpec((_ROWS_PER_PROG, 4, _S, _S),
                         lambda b: (b, 0, 0, 0)),
            pl.BlockSpec((_ROWS_PER_PROG, 2, _S, _S),
                         lambda b: (b, 0, 0, 0)),
            pl.BlockSpec((_ROWS_PER_PROG, 4, _S, _S),
                         lambda b: (b, 0, 0, 0)),
            pl.BlockSpec((_ROWS_PER_PROG, _NUM_CLASSES, _S, _S),
                         lambda b: (b, 0, 0, 0)),
            pl.BlockSpec((4, _S, _S), lambda b: (0, 0, 0)),
            pl.BlockSpec((_ROWS_PER_PROG, _NUM_OBJ, 5), lambda b: (b, 0, 0)),
        ],
        out_specs=pl.BlockSpec((_ROWS_PER_PROG, 8, 128), lambda b: (b, 0, 0)),
        out_shape=jax.ShapeDtypeStruct((B, 8, 128), jnp.float32),
        compiler_params=pltpu.CompilerParams(
            dimension_semantics=("parallel",)),
    )(_to_tiles(arm_loc_pred), _to_tiles(arm_conf_pred),
      _to_tiles(odm_loc_pred), _to_tiles(odm_conf_pred), priors_t, targets)

    loss_l = jnp.sum(out[:, 0, 0])
    loss_c = jnp.sum(out[:, 0, 1])
    total = jnp.sum(out[:, 0, 2])
    return (loss_l / total, loss_c / total)


# speedup vs baseline: 45.5224x; 1.3199x over previous
import jax
import jax.numpy as jnp
from jax.experimental import pallas as pl
from jax.experimental.pallas import tpu as pltpu

_NUM_CLASSES = 21
_OVERLAP_THRESH = 0.5
_NEG_POS_RATIO = 3
_POS_PRIOR_THRESHOLD = 0.01
_NUM_OBJ = 8
_BISECT_ITERS = 18
_S = 128
_P_PAD = _S * _S
_R = 2


def _pair_kernel(arm_loc_ref, arm_conf_ref, odm_loc_ref, odm_conf_ref,
                 priors_ref, targets_ref, valid_p, out_ref):
    T = (_S, _S)
    R = range(_R)

    al = arm_loc_ref[:]
    pcx = priors_ref[0]
    pcy = priors_ref[1]
    pw = priors_ref[2]
    ph = priors_ref[3]
    cx = pcx + al[:, 0] * (0.1 * pw)
    cy = pcy + al[:, 1] * (0.1 * ph)
    w = pw * jnp.exp(al[:, 2] * 0.2)
    h = ph * jnp.exp(al[:, 3] * 0.2)
    rx1 = cx - w * 0.5
    ry1 = cy - h * 0.5
    rx2 = cx + w * 0.5
    ry2 = cy + h * 0.5
    area_p = w * h

    idx2d = (jax.lax.broadcasted_iota(jnp.int32, T, 0) * _S
             + jax.lax.broadcasted_iota(jnp.int32, T, 1))

    tgt = targets_ref[:]

    bto = jnp.zeros((_R,) + T, jnp.float32)
    bti = jnp.zeros((_R,) + T, jnp.int32)
    best_prior = []
    for i in range(_NUM_OBJ):
        tx1 = tgt[:, i:i + 1, 0:1]
        ty1 = tgt[:, i:i + 1, 1:2]
        tx2 = tgt[:, i:i + 1, 2:3]
        ty2 = tgt[:, i:i + 1, 3:4]
        ix = jnp.maximum(jnp.minimum(tx2, rx2) - jnp.maximum(tx1, rx1), 0.0)
        iy = jnp.maximum(jnp.minimum(ty2, ry2) - jnp.maximum(ty1, ry1), 0.0)
        inter = ix * iy
        area_t = (tx2 - tx1) * (ty2 - ty1)
        ov = inter / (area_t + area_p - inter)
        if i == 0:
            bto = ov
        else:
            upd = ov > bto
            bto = jnp.maximum(bto, ov)
            bti = jnp.where(upd, i, bti)
        m_i = [jnp.max(ov[r]) for r in R]
        best_prior.append([jnp.min(jnp.where(ov[r] == m_i[r], idx2d, _P_PAD))
                           for r in R])

    for i in range(_NUM_OBJ):
        hit = jnp.stack([idx2d == best_prior[i][r] for r in R])
        bto = jnp.where(hit, 2.0, bto)
        bti = jnp.where(hit, i, bti)

    mx1 = jnp.zeros((_R,) + T, jnp.float32)
    my1 = jnp.zeros((_R,) + T, jnp.float32)
    mx2 = jnp.zeros((_R,) + T, jnp.float32)
    my2 = jnp.zeros((_R,) + T, jnp.float32)
    lab = jnp.zeros((_R,) + T, jnp.float32)
    for i in range(_NUM_OBJ):
        sel = bti == i
        mx1 = jnp.where(sel, tgt[:, i:i + 1, 0:1], mx1)
        my1 = jnp.where(sel, tgt[:, i:i + 1, 1:2], my1)
        mx2 = jnp.where(sel, tgt[:, i:i + 1, 2:3], mx2)
        my2 = jnp.where(sel, tgt[:, i:i + 1, 3:4], my2)
        lab = jnp.where(sel, tgt[:, i:i + 1, 4:5], lab)

    conf = jnp.where(bto < _OVERLAP_THRESH, 0.0, lab)
    conf_i = conf.astype(jnp.int32)
    pos = conf_i > 0

    g_cx = ((mx1 + mx2) * 0.5 - cx) / (0.1 * w)
    g_cy = ((my1 + my2) * 0.5 - cy) / (0.1 * h)
    g_w = jnp.log((mx2 - mx1) / w) / 0.2
    g_h = jnp.log((my2 - my1) / h) / 0.2
    ol = odm_loc_ref[:]

    def _sl1(d):
        ad = jnp.abs(d)
        return jnp.where(ad < 1.0, 0.5 * d * d, ad - 0.5)

    sl1 = (_sl1(ol[:, 0] - g_cx) + _sl1(ol[:, 1] - g_cy)
           + _sl1(ol[:, 2] - g_w) + _sl1(ol[:, 3] - g_h))
    sl1 = jnp.where(pos, sl1, 0.0)
    loss_l = [jnp.sum(sl1[r]) for r in R]

    oc = odm_conf_ref[:]
    mx = jnp.max(oc, axis=1)
    lse = mx + jnp.log(jnp.sum(jnp.exp(oc - mx[:, None]), axis=1))
    iota_c = jax.lax.broadcasted_iota(jnp.int32, oc.shape, 1)
    gathered = jnp.sum(jnp.where(iota_c == conf_i[:, None], oc, 0.0), axis=1)
    ce = lse - gathered

    ac = arm_conf_ref[:]
    s1 = jax.nn.sigmoid(ac[:, 1] - ac[:, 0])
    keep = jnp.logical_or(pos, jnp.logical_and(conf_i <= 0,
                                               s1 < _POS_PRIOR_THRESHOLD))
    proxy = jnp.where(jnp.logical_or(keep, jnp.logical_not(valid_p)),
                      0.0, ce)

    posf = pos.astype(jnp.float32)
    num_pos = [jnp.sum(posf[r]) for r in R]
    negf = (proxy > 0.0).astype(jnp.float32)
    max_neg = [jnp.sum(negf[r]) for r in R]
    k = [jnp.minimum(_NEG_POS_RATIO * num_pos[r], max_neg[r]) for r in R]

    lo = [jnp.float32(0.0) for _ in R]
    hi = [jnp.max(proxy[r]) for r in R]
    for _ in range(_BISECT_ITERS):
        mid = [0.5 * (lo[r] + hi[r]) for r in R]
        cnt = [jnp.sum((proxy[r] >= mid[r]).astype(jnp.float32)) for r in R]
        ok = [cnt[r] >= k[r] for r in R]
        lo = [jnp.where(ok[r], mid[r], lo[r]) for r in R]
        hi = [jnp.where(ok[r], hi[r], mid[r]) for r in R]

    ce_pos = jnp.where(pos, ce, 0.0)
    for r in R:
        ge = proxy[r] >= lo[r]
        cnt_lo = jnp.sum(ge.astype(jnp.float32))
        sum_lo = jnp.sum(jnp.where(ge, proxy[r], 0.0))
        topk = sum_lo - (cnt_lo - k[r]) * lo[r]
        topk = jnp.where(k[r] > 0.0, topk, 0.0)
        loss_c = jnp.sum(ce_pos[r]) + topk

        rr = jax.lax.broadcasted_iota(jnp.int32, (8, 128), 0)
        cc = jax.lax.broadcasted_iota(jnp.int32, (8, 128), 1)
        first = rr == 0
        tile = jnp.where(jnp.logical_and(first, cc == 0), loss_l[r], 0.0)
        tile = jnp.where(jnp.logical_and(first, cc == 1), loss_c, tile)
        tile = jnp.where(jnp.logical_and(first, cc == 2), num_pos[r], tile)
        out_ref[r] = tile


def _to_tiles(x):
    B, P, C = x.shape
    xt = jnp.transpose(x, (0, 2, 1))
    xt = jnp.pad(xt, ((0, 0), (0, 0), (0, _P_PAD - P)))
    return xt.reshape(B, C, _S, _S)


@jax.jit
def kernel(arm_loc_pred, arm_conf_pred, odm_loc_pred, odm_conf_pred, priors,
           targets):
    B, P, _ = arm_loc_pred.shape
    priors_t = jnp.pad(jnp.transpose(priors, (1, 0)),
                       ((0, 0), (0, _P_PAD - P))).reshape(4, _S, _S)

    def body(*refs):
        P_idx = (jax.lax.broadcasted_iota(jnp.int32, (_S, _S), 0) * _S
                 + jax.lax.broadcasted_iota(jnp.int32, (_S, _S), 1))
        _pair_kernel(*refs[:6], P_idx < P, refs[6])

    out = pl.pallas_call(
        body,
        grid=(B // _R,),
        in_specs=[
            pl.BlockSpec((_R, 4, _S, _S), lambda b: (b, 0, 0, 0)),
            pl.BlockSpec((_R, 2, _S, _S), lambda b: (b, 0, 0, 0)),
            pl.BlockSpec((_R, 4, _S, _S), lambda b: (b, 0, 0, 0)),
            pl.BlockSpec((_R, _NUM_CLASSES, _S, _S), lambda b: (b, 0, 0, 0)),
            pl.BlockSpec((4, _S, _S), lambda b: (0, 0, 0)),
            pl.BlockSpec((_R, _NUM_OBJ, 5), lambda b: (b, 0, 0)),
        ],
        out_specs=pl.BlockSpec((_R, 8, 128), lambda b: (b, 0, 0)),
        out_shape=jax.ShapeDtypeStruct((B, 8, 128), jnp.float32),
        compiler_params=pltpu.CompilerParams(
            dimension_semantics=("parallel",)),
    )(_to_tiles(arm_loc_pred), _to_tiles(arm_conf_pred),
      _to_tiles(odm_loc_pred), _to_tiles(odm_conf_pred), priors_t, targets)

    loss_l = jnp.sum(out[:, 0, 0])
    loss_c = jnp.sum(out[:, 0, 1])
    total = jnp.sum(out[:, 0, 2])
    return (loss_l / total, loss_c / total)


# speedup vs baseline: 53.5156x; 1.1756x over previous
import jax
import jax.numpy as jnp
from jax.experimental import pallas as pl
from jax.experimental.pallas import tpu as pltpu

_NUM_CLASSES = 21
_OVERLAP_THRESH = 0.5
_NEG_POS_RATIO = 3
_POS_PRIOR_THRESHOLD = 0.01
_NUM_OBJ = 8
_BISECT_ITERS = 18
_S = 128
_P_PAD = _S * _S
_R = 4


def _pair_kernel(arm_loc_ref, arm_conf_ref, odm_loc_ref, odm_conf_ref,
                 priors_ref, targets_ref, valid_p, out_ref):
    T = (_S, _S)
    R = range(_R)

    al = arm_loc_ref[:]
    pcx = priors_ref[0]
    pcy = priors_ref[1]
    pw = priors_ref[2]
    ph = priors_ref[3]
    cx = pcx + al[:, 0] * (0.1 * pw)
    cy = pcy + al[:, 1] * (0.1 * ph)
    w = pw * jnp.exp(al[:, 2] * 0.2)
    h = ph * jnp.exp(al[:, 3] * 0.2)
    rx1 = cx - w * 0.5
    ry1 = cy - h * 0.5
    rx2 = cx + w * 0.5
    ry2 = cy + h * 0.5
    area_p = w * h

    idx2d = (jax.lax.broadcasted_iota(jnp.int32, T, 0) * _S
             + jax.lax.broadcasted_iota(jnp.int32, T, 1))

    tgt = targets_ref[:]

    bto = jnp.zeros((_R,) + T, jnp.float32)
    bti = jnp.zeros((_R,) + T, jnp.int32)
    best_prior = []
    for i in range(_NUM_OBJ):
        tx1 = tgt[:, i:i + 1, 0:1]
        ty1 = tgt[:, i:i + 1, 1:2]
        tx2 = tgt[:, i:i + 1, 2:3]
        ty2 = tgt[:, i:i + 1, 3:4]
        ix = jnp.maximum(jnp.minimum(tx2, rx2) - jnp.maximum(tx1, rx1), 0.0)
        iy = jnp.maximum(jnp.minimum(ty2, ry2) - jnp.maximum(ty1, ry1), 0.0)
        inter = ix * iy
        area_t = (tx2 - tx1) * (ty2 - ty1)
        ov = inter / (area_t + area_p - inter)
        if i == 0:
            bto = ov
        else:
            upd = ov > bto
            bto = jnp.maximum(bto, ov)
            bti = jnp.where(upd, i, bti)
        m_i = [jnp.max(ov[r]) for r in R]
        best_prior.append([jnp.min(jnp.where(ov[r] == m_i[r], idx2d, _P_PAD))
                           for r in R])

    for i in range(_NUM_OBJ):
        hit = jnp.stack([idx2d == best_prior[i][r] for r in R])
        bto = jnp.where(hit, 2.0, bto)
        bti = jnp.where(hit, i, bti)

    mx1 = jnp.zeros((_R,) + T, jnp.float32)
    my1 = jnp.zeros((_R,) + T, jnp.float32)
    mx2 = jnp.zeros((_R,) + T, jnp.float32)
    my2 = jnp.zeros((_R,) + T, jnp.float32)
    lab = jnp.zeros((_R,) + T, jnp.float32)
    for i in range(_NUM_OBJ):
        sel = bti == i
        mx1 = jnp.where(sel, tgt[:, i:i + 1, 0:1], mx1)
        my1 = jnp.where(sel, tgt[:, i:i + 1, 1:2], my1)
        mx2 = jnp.where(sel, tgt[:, i:i + 1, 2:3], mx2)
        my2 = jnp.where(sel, tgt[:, i:i + 1, 3:4], my2)
        lab = jnp.where(sel, tgt[:, i:i + 1, 4:5], lab)

    conf = jnp.where(bto < _OVERLAP_THRESH, 0.0, lab)
    conf_i = conf.astype(jnp.int32)
    pos = conf_i > 0

    g_cx = ((mx1 + mx2) * 0.5 - cx) / (0.1 * w)
    g_cy = ((my1 + my2) * 0.5 - cy) / (0.1 * h)
    g_w = jnp.log((mx2 - mx1) / w) / 0.2
    g_h = jnp.log((my2 - my1) / h) / 0.2
    ol = odm_loc_ref[:]

    def _sl1(d):
        ad = jnp.abs(d)
        return jnp.where(ad < 1.0, 0.5 * d * d, ad - 0.5)

    sl1 = (_sl1(ol[:, 0] - g_cx) + _sl1(ol[:, 1] - g_cy)
           + _sl1(ol[:, 2] - g_w) + _sl1(ol[:, 3] - g_h))
    sl1 = jnp.where(pos, sl1, 0.0)
    loss_l = [jnp.sum(sl1[r]) for r in R]

    oc = odm_conf_ref[:]
    mx = jnp.max(oc, axis=1)
    lse = mx + jnp.log(jnp.sum(jnp.exp(oc - mx[:, None]), axis=1))
    iota_c = jax.lax.broadcasted_iota(jnp.int32, oc.shape, 1)
    gathered = jnp.sum(jnp.where(iota_c == conf_i[:, None], oc, 0.0), axis=1)
    ce = lse - gathered

    ac = arm_conf_ref[:]
    s1 = jax.nn.sigmoid(ac[:, 1] - ac[:, 0])
    keep = jnp.logical_or(pos, jnp.logical_and(conf_i <= 0,
                                               s1 < _POS_PRIOR_THRESHOLD))
    proxy = jnp.where(jnp.logical_or(keep, jnp.logical_not(valid_p)),
                      0.0, ce)

    posf = pos.astype(jnp.float32)
    num_pos = [jnp.sum(posf[r]) for r in R]
    negf = (proxy > 0.0).astype(jnp.float32)
    max_neg = [jnp.sum(negf[r]) for r in R]
    k = [jnp.minimum(_NEG_POS_RATIO * num_pos[r], max_neg[r]) for r in R]

    lo = [jnp.float32(0.0) for _ in R]
    hi = [jnp.max(proxy[r]) for r in R]
    for _ in range(_BISECT_ITERS):
        mid = [0.5 * (lo[r] + hi[r]) for r in R]
        cnt = [jnp.sum((proxy[r] >= mid[r]).astype(jnp.float32)) for r in R]
        ok = [cnt[r] >= k[r] for r in R]
        lo = [jnp.where(ok[r], mid[r], lo[r]) for r in R]
        hi = [jnp.where(ok[r], hi[r], mid[r]) for r in R]

    ce_pos = jnp.where(pos, ce, 0.0)
    for r in R:
        ge = proxy[r] >= lo[r]
        cnt_lo = jnp.sum(ge.astype(jnp.float32))
        sum_lo = jnp.sum(jnp.where(ge, proxy[r], 0.0))
        topk = sum_lo - (cnt_lo - k[r]) * lo[r]
        topk = jnp.where(k[r] > 0.0, topk, 0.0)
        loss_c = jnp.sum(ce_pos[r]) + topk

        rr = jax.lax.broadcasted_iota(jnp.int32, (8, 128), 0)
        cc = jax.lax.broadcasted_iota(jnp.int32, (8, 128), 1)
        first = rr == 0
        tile = jnp.where(jnp.logical_and(first, cc == 0), loss_l[r], 0.0)
        tile = jnp.where(jnp.logical_and(first, cc == 1), loss_c, tile)
        tile = jnp.where(jnp.logical_and(first, cc == 2), num_pos[r], tile)
        out_ref[r] = tile


def _to_tiles(x):
    B, P, C = x.shape
    xt = jnp.transpose(x, (0, 2, 1))
    xt = jnp.pad(xt, ((0, 0), (0, 0), (0, _P_PAD - P)))
    return xt.reshape(B, C, _S, _S)


@jax.jit
def kernel(arm_loc_pred, arm_conf_pred, odm_loc_pred, odm_conf_pred, priors,
           targets):
    B, P, _ = arm_loc_pred.shape
    priors_t = jnp.pad(jnp.transpose(priors, (1, 0)),
                       ((0, 0), (0, _P_PAD - P))).reshape(4, _S, _S)

    def body(*refs):
        P_idx = (jax.lax.broadcasted_iota(jnp.int32, (_S, _S), 0) * _S
                 + jax.lax.broadcasted_iota(jnp.int32, (_S, _S), 1))
        _pair_kernel(*refs[:6], P_idx < P, refs[6])

    out = pl.pallas_call(
        body,
        grid=(B // _R,),
        in_specs=[
            pl.BlockSpec((_R, 4, _S, _S), lambda b: (b, 0, 0, 0)),
            pl.BlockSpec((_R, 2, _S, _S), lambda b: (b, 0, 0, 0)),
            pl.BlockSpec((_R, 4, _S, _S), lambda b: (b, 0, 0, 0)),
            pl.BlockSpec((_R, _NUM_CLASSES, _S, _S), lambda b: (b, 0, 0, 0)),
            pl.BlockSpec((4, _S, _S), lambda b: (0, 0, 0)),
            pl.BlockSpec((_R, _NUM_OBJ, 5), lambda b: (b, 0, 0)),
        ],
        out_specs=pl.BlockSpec((_R, 8, 128), lambda b: (b, 0, 0)),
        out_shape=jax.ShapeDtypeStruct((B, 8, 128), jnp.float32),
        compiler_params=pltpu.CompilerParams(
            dimension_semantics=("parallel",)),
    )(_to_tiles(arm_loc_pred), _to_tiles(arm_conf_pred),
      _to_tiles(odm_loc_pred), _to_tiles(odm_conf_pred), priors_t, targets)

    loss_l = jnp.sum(out[:, 0, 0])
    loss_c = jnp.sum(out[:, 0, 1])
    total = jnp.sum(out[:, 0, 2])
    return (loss_l / total, loss_c / total)
